# baseline (speedup 1.0000x reference)
import jax
import jax.numpy as jnp
from jax.experimental import pallas as pl
from jax.experimental.pallas import tpu as pltpu


def _fused_kernel(x_ref, w_ref, nb_ref, out_ref, x1_ref):
    @pl.when(pl.program_id(0) == 0)
    def _():
        x1 = jax.lax.dot_general(
            x_ref[...], w_ref[...],
            (((1,), (1,)), ((), ())),
            preferred_element_type=jnp.float32,
        )
        x1_ref[...] = x1.astype(jnp.bfloat16)

    out_ref[...] = jax.lax.dot(
        nb_ref[...].astype(jnp.bfloat16), x1_ref[...],
        preferred_element_type=jnp.float32,
    )


def kernel(x, neighborhood, W):
    n, d_in = x.shape
    d_out = W.shape[0]
    bm = 200
    assert n % bm == 0
    grid = (n // bm,)
    return pl.pallas_call(
        _fused_kernel,
        grid=grid,
        in_specs=[
            pl.BlockSpec((n, d_in), lambda i: (0, 0)),
            pl.BlockSpec((d_out, d_in), lambda i: (0, 0)),
            pl.BlockSpec((bm, n), lambda i: (i, 0)),
        ],
        out_specs=pl.BlockSpec((bm, d_out), lambda i: (i, 0)),
        out_shape=jax.ShapeDtypeStruct((n, d_out), jnp.float32),
        scratch_shapes=[pltpu.VMEM((n, d_out), jnp.bfloat16)],
        compiler_params=pltpu.CompilerParams(
            dimension_semantics=("arbitrary",),
        ),
    )(x, W, neighborhood)


# speedup vs baseline: 1.0085x; 1.0085x over previous
import jax
import jax.numpy as jnp
from jax.experimental import pallas as pl
from jax.experimental.pallas import tpu as pltpu


def _fused_kernel(x_ref, w_ref, nb_ref, out_ref, x1_ref):
    @pl.when(pl.program_id(0) == 0)
    def _():
        x1 = jax.lax.dot_general(
            x_ref[...], w_ref[...],
            (((1,), (1,)), ((), ())),
            preferred_element_type=jnp.float32,
        )
        x1_ref[...] = x1.astype(jnp.bfloat16)

    out_ref[...] = jax.lax.dot(
        nb_ref[...].astype(jnp.bfloat16), x1_ref[...],
        preferred_element_type=jnp.float32,
    )


def kernel(x, neighborhood, W):
    n, d_in = x.shape
    d_out = W.shape[0]
    bm = 400
    assert n % bm == 0
    grid = (n // bm,)
    return pl.pallas_call(
        _fused_kernel,
        grid=grid,
        in_specs=[
            pl.BlockSpec((n, d_in), lambda i: (0, 0)),
            pl.BlockSpec((d_out, d_in), lambda i: (0, 0)),
            pl.BlockSpec((bm, n), lambda i: (i, 0)),
        ],
        out_specs=pl.BlockSpec((bm, d_out), lambda i: (i, 0)),
        out_shape=jax.ShapeDtypeStruct((n, d_out), jnp.float32),
        scratch_shapes=[pltpu.VMEM((n, d_out), jnp.bfloat16)],
        compiler_params=pltpu.CompilerParams(
            dimension_semantics=("arbitrary",),
        ),
    )(x, W, neighborhood)
